# Initial kernel scaffold; baseline (speedup 1.0000x reference)
import jax
import jax.numpy as jnp
from jax.experimental import pallas as pl


def kernel(token_ids, lut):
    raise NotImplementedError("write your pallas kernel here")


# speedup vs baseline: 1.1091x; 1.1091x over previous
import functools

import jax
import jax.numpy as jnp
from jax import lax
from jax.experimental import pallas as pl
from jax.experimental.pallas import tpu as pltpu
from jax.experimental.pallas import tpu_sc as plsc

NC = 2
NS = 16
NW = NC * NS
CH = 128
G = 20


def _emb_body(idx_hbm, table_hbm, out_hbm, idx_v, rows_v, sem):
    n_chunks = idx_hbm.shape[0] // CH
    chunks_per_w = n_chunks // NW
    wid = lax.axis_index("s") * NC + lax.axis_index("c")
    row0 = wid * chunks_per_w

    def outer(g, carry):
        base = (row0 + g * G) * CH
        pltpu.sync_copy(idx_hbm.at[pl.ds(base, G * CH)], idx_v)
        copies = [
            pltpu.async_copy(
                table_hbm.at[idx_v.at[pl.ds(j * CH, CH)]],
                rows_v.at[pl.ds(j * CH, CH)],
                sem,
            )
            for j in range(G)
        ]
        for cp in copies:
            cp.wait()
        pltpu.sync_copy(rows_v, out_hbm.at[pl.ds(base, G * CH)])
        return carry

    lax.fori_loop(0, chunks_per_w // G, outer, 0)


def kernel(token_ids, lut):
    n, s = token_ids.shape
    v, d = lut.shape
    b = n * s
    idx = token_ids.reshape(b).astype(jnp.int32)
    mesh = plsc.VectorSubcoreMesh(core_axis_name="c", subcore_axis_name="s")
    k = pl.kernel(
        _emb_body,
        mesh=mesh,
        out_type=jax.ShapeDtypeStruct((b, d), jnp.float32),
        compiler_params=pltpu.CompilerParams(use_tc_tiling_on_sc=False),
        scratch_types=[
            pltpu.VMEM((G * CH,), jnp.int32),
            pltpu.VMEM((G * CH, d), jnp.float32),
            pltpu.SemaphoreType.DMA,
        ],
    )
    out = k(idx, lut)
    return out.reshape(n, s, d)
